# Initial kernel scaffold; baseline (speedup 1.0000x reference)
import jax
import jax.numpy as jnp
from jax.experimental import pallas as pl


def kernel(q, k_buffer, v_buffer, Wk, Wv):
    raise NotImplementedError("write your pallas kernel here")


# speedup vs baseline: 2.1924x; 2.1924x over previous
import functools

import jax
import jax.numpy as jnp
from jax import lax
from jax.experimental import pallas as pl
from jax.experimental.pallas import tpu as pltpu

NUM_Q_HEADS = 32
NUM_KV_HEADS = 2
D = 128
KS = 32
STRIDE = 16
SELECT = 64
TOP_N = 16
N_INIT = 1
N_LOCAL = 2
GROUP = NUM_Q_HEADS // NUM_KV_HEADS
SM_SCALE = D ** (-0.5)
S = 4096
R = S // STRIDE
CP = 256
NB = S // SELECT
NEG = -1e30

SB = 256


def _compress_body(x_ref, xs_ref, w_ref, o_ref, h):
    x = x_ref[h]
    xs = xs_ref[h]
    w1 = w_ref[:, : STRIDE * D]
    w2 = w_ref[:, STRIDE * D:]
    gate = lax.dot_general(x, w1, (((1,), (1,)), ((), ())))
    gate = gate + lax.dot_general(xs, w2, (((1,), (1,)), ((), ())))
    mx = jnp.max(gate, axis=1, keepdims=True)
    e = jnp.exp(gate - mx)
    wts = e / jnp.sum(e, axis=1, keepdims=True)
    acc = jnp.zeros((R, D), jnp.float32)
    for j in range(STRIDE):
        acc = acc + wts[:, j:j + 1] * x[:, j * D:(j + 1) * D]
        acc = acc + wts[:, j + STRIDE:j + STRIDE + 1] * xs[:, j * D:(j + 1) * D]
    o_ref[h] = acc


def _compress_kernel(kb_ref, kbs_ref, vb_ref, vbs_ref, wk_ref, wv_ref,
                     ck_ref, cv_ref):
    for h in range(NUM_KV_HEADS):
        _compress_body(kb_ref, kbs_ref, wk_ref, ck_ref, h)
        _compress_body(vb_ref, vbs_ref, wv_ref, cv_ref, h)


def _attn_kernel(q_ref, ck_ref, cv_ref, o_ref, topk_ref):
    i = pl.program_id(0)
    t = i * SB + lax.broadcasted_iota(jnp.int32, (SB, CP), 0)
    c = lax.broadcasted_iota(jnp.int32, (SB, CP), 1)
    valid = (t >= c * STRIDE + KS - 1) & (c < CP - 1)
    mask = valid | ((t < KS - 1) & (c == 0))

    ci = lax.broadcasted_iota(jnp.int32, (CP, NB), 0)
    bi = lax.broadcasted_iota(jnp.int32, (CP, NB), 1)
    agg = ((ci // (SELECT // STRIDE)) == bi).astype(jnp.float32)

    qb = (i * SB + lax.broadcasted_iota(jnp.int32, (SB, NB), 0)) // SELECT
    b2 = lax.broadcasted_iota(jnp.int32, (SB, NB), 1)
    forced = (b2 < N_INIT) | ((b2 <= qb) & (b2 >= qb - (N_LOCAL - 1)))
    causal = b2 <= qb

    for h in range(NUM_KV_HEADS):
        ckh = ck_ref[h]
        cvh = cv_ref[h]
        p = jnp.zeros((SB, CP), jnp.float32)
        for g in range(GROUP):
            hq = h * GROUP + g
            qg = q_ref[:, hq * D:(hq + 1) * D]
            s = lax.dot_general(qg, ckh, (((1,), (1,)), ((), ()))) * SM_SCALE
            s = jnp.where(mask, s, NEG)
            mx = jnp.max(s, axis=1, keepdims=True)
            e = jnp.exp(s - mx)
            probs = e / jnp.sum(e, axis=1, keepdims=True)
            p = p + probs
            o_ref[:, hq * D:(hq + 1) * D] = jnp.dot(probs, cvh)
        slc = jnp.dot(p, agg)
        slc = jnp.where(forced, jnp.float32(1e9), slc)
        slc = jnp.where(causal, slc, NEG)

        lane = lax.broadcasted_iota(jnp.int32, (SB, NB), 1)
        work = slc
        cols = []
        for _ in range(TOP_N):
            mx = jnp.max(work, axis=1, keepdims=True)
            idx = jnp.min(jnp.where(work == mx, lane, NB), axis=1,
                          keepdims=True)
            cols.append(idx)
            work = jnp.where(lane == idx, -jnp.inf, work)
        topk_ref[:, h, :] = jnp.concatenate(cols, axis=1)


def kernel(q, k_buffer, v_buffer, Wk, Wv):
    kb = jnp.transpose(k_buffer, (1, 0, 2)).reshape(NUM_KV_HEADS, R, STRIDE * D)
    vb = jnp.transpose(v_buffer, (1, 0, 2)).reshape(NUM_KV_HEADS, R, STRIDE * D)
    pad = jnp.zeros((NUM_KV_HEADS, 1, STRIDE * D), jnp.float32)
    kbs = jnp.concatenate([kb[:, 1:], pad], axis=1)
    vbs = jnp.concatenate([vb[:, 1:], pad], axis=1)

    ck, cv = pl.pallas_call(
        _compress_kernel,
        out_shape=[
            jax.ShapeDtypeStruct((NUM_KV_HEADS, R, D), jnp.float32),
            jax.ShapeDtypeStruct((NUM_KV_HEADS, R, D), jnp.float32),
        ],
    )(kb, kbs, vb, vbs, Wk, Wv)

    o, topk = pl.pallas_call(
        _attn_kernel,
        grid=(S // SB,),
        in_specs=[
            pl.BlockSpec((SB, NUM_Q_HEADS * D), lambda i: (i, 0)),
            pl.BlockSpec((NUM_KV_HEADS, CP, D), lambda i: (0, 0, 0)),
            pl.BlockSpec((NUM_KV_HEADS, CP, D), lambda i: (0, 0, 0)),
        ],
        out_specs=[
            pl.BlockSpec((SB, NUM_Q_HEADS * D), lambda i: (i, 0)),
            pl.BlockSpec((SB, NUM_KV_HEADS, TOP_N), lambda i: (i, 0, 0)),
        ],
        out_shape=[
            jax.ShapeDtypeStruct((S, NUM_Q_HEADS * D), jnp.float32),
            jax.ShapeDtypeStruct((S, NUM_KV_HEADS, TOP_N), jnp.int32),
        ],
    )(q, ck, cv)
    return o, topk


# speedup vs baseline: 2.6407x; 1.2045x over previous
import jax
import jax.numpy as jnp
from jax import lax
from jax.experimental import pallas as pl
from jax.experimental.pallas import tpu as pltpu
from jax.experimental.pallas import tpu_sc as plsc

NUM_Q_HEADS = 32
NUM_KV_HEADS = 2
D = 128
KS = 32
STRIDE = 16
SELECT = 64
TOP_N = 16
N_INIT = 1
N_LOCAL = 2
GROUP = NUM_Q_HEADS // NUM_KV_HEADS
SM_SCALE = D ** (-0.5)
S = 4096
R = S // STRIDE
CP = 256
NB = S // SELECT
NEG = -1e30

SB = 256

SC_NC = 2
SC_NS = 16
SC_NW = SC_NC * SC_NS
SC_ROWS = (S * NUM_KV_HEADS) // SC_NW

BF = jnp.bfloat16


def _bdot(a, b):
    return lax.dot_general(a.astype(BF), b.astype(BF),
                           (((1,), (0,)), ((), ())),
                           preferred_element_type=jnp.float32)


def _compress_body(b_ref, w_ref, o_ref, h):
    xs = []
    g1 = jnp.zeros((R, KS), jnp.float32)
    g2 = jnp.zeros((R, KS), jnp.float32)
    for j in range(STRIDE):
        x = b_ref[:, j, h, :]
        xs.append(x)
        xb = x.astype(BF)
        w1j = w_ref[:, j * D:(j + 1) * D].astype(BF)
        w2j = w_ref[:, (j + STRIDE) * D:(j + STRIDE + 1) * D].astype(BF)
        g1 = g1 + lax.dot_general(xb, w1j, (((1,), (1,)), ((), ())),
                                  preferred_element_type=jnp.float32)
        g2 = g2 + lax.dot_general(xb, w2j, (((1,), (1,)), ((), ())),
                                  preferred_element_type=jnp.float32)
    gate = g1 + pltpu.roll(g2, R - 1, 0)
    mx = jnp.max(gate, axis=1, keepdims=True)
    e = jnp.exp(gate - mx)
    wts = e / jnp.sum(e, axis=1, keepdims=True)
    w2r = pltpu.roll(wts[:, STRIDE:], 1, 0)

    acc = jnp.zeros((R, D), jnp.float32)
    y = jnp.zeros((R, D), jnp.float32)
    for j in range(STRIDE):
        acc = acc + wts[:, j:j + 1] * xs[j]
        y = y + w2r[:, j:j + 1] * xs[j]
    o_ref[h] = acc + pltpu.roll(y, R - 1, 0)


def _compress_kernel(kb_ref, vb_ref, wk_ref, wv_ref, ck_ref, cv_ref):
    for h in range(NUM_KV_HEADS):
        _compress_body(kb_ref, wk_ref, ck_ref, h)
        _compress_body(vb_ref, wv_ref, cv_ref, h)


def _attn_kernel(q_ref, ck_ref, cv_ref, o_ref, slc_ref):
    i = pl.program_id(0)
    t = i * SB + lax.broadcasted_iota(jnp.int32, (SB, CP), 0)
    c = lax.broadcasted_iota(jnp.int32, (SB, CP), 1)
    valid = (t >= c * STRIDE + KS - 1) & (c < CP - 1)
    mask = valid | ((t < KS - 1) & (c == 0))

    ci = lax.broadcasted_iota(jnp.int32, (CP, NB), 0)
    bi = lax.broadcasted_iota(jnp.int32, (CP, NB), 1)
    agg = ((ci // (SELECT // STRIDE)) == bi).astype(jnp.float32)

    qb = (i * SB + lax.broadcasted_iota(jnp.int32, (SB, NB), 0)) // SELECT
    b2 = lax.broadcasted_iota(jnp.int32, (SB, NB), 1)
    forced = (b2 < N_INIT) | ((b2 <= qb) & (b2 >= qb - (N_LOCAL - 1)))
    causal = b2 <= qb
    b2f = b2.astype(jnp.float32)
    forced_vals = 1e9 - b2f * 1024.0
    masked_vals = NEG - b2f * 4e24

    for h in range(NUM_KV_HEADS):
        ckh = ck_ref[h].astype(BF)
        cvh = cv_ref[h].astype(BF)
        p = jnp.zeros((SB, CP), jnp.float32)
        for g in range(GROUP):
            hq = h * GROUP + g
            qg = q_ref[:, hq * D:(hq + 1) * D].astype(BF)
            s = lax.dot_general(qg, ckh, (((1,), (1,)), ((), ())),
                                preferred_element_type=jnp.float32)
            s = jnp.where(mask, s * SM_SCALE, NEG)
            m = jnp.max(s, axis=1, keepdims=True)
            e = jnp.exp(s - m)
            probs = e / jnp.sum(e, axis=1, keepdims=True)
            o_ref[:, hq * D:(hq + 1) * D] = _bdot(probs, cvh)
            p = p + probs
        slc = _bdot(p, agg)
        slc = jnp.where(forced, forced_vals, slc)
        slc = jnp.where(causal, slc, masked_vals)
        bits = pltpu.bitcast(slc, jnp.int32)
        okey = jnp.where(bits < 0, bits ^ jnp.int32(0x7FFFFFFF), bits)
        slc_ref[:, h * NB:(h + 1) * NB] = okey - b2


def _topk_sc_body(slc_hbm, out_hbm, slc_v, out_v):
    wid = lax.axis_index("s") * SC_NC + lax.axis_index("c")
    base = wid * SC_ROWS
    pltpu.sync_copy(slc_hbm.at[pl.ds(base, SC_ROWS)], slc_v)
    lane = lax.broadcasted_iota(jnp.int32, (16,), 0)

    def merge(a, b):
        ak, ai = a
        bk = lax.rev(b[0], (0,))
        bi = lax.rev(b[1], (0,))
        take_a = ak >= bk
        mk = jnp.where(take_a, ak, bk)
        mi = jnp.where(take_a, ai, bi)
        return plsc.sort_key_val(mk, mi, descending=True)

    def body(rr, _):
        runs = []
        for j in range(NB // 16):
            k = slc_v[rr, pl.ds(16 * j, 16)]
            runs.append(plsc.sort_key_val(k, lane + 16 * j, descending=True))
        mk, mi = merge(merge(runs[0], runs[1]), merge(runs[2], runs[3]))
        out_v[rr] = mi
        return 0

    lax.fori_loop(0, SC_ROWS, body, 0)
    pltpu.sync_copy(out_v, out_hbm.at[pl.ds(base, SC_ROWS)])


def _make_topk_sc():
    mesh = plsc.VectorSubcoreMesh(core_axis_name="c", subcore_axis_name="s")
    return pl.kernel(
        _topk_sc_body,
        mesh=mesh,
        out_type=jax.ShapeDtypeStruct((S * NUM_KV_HEADS, TOP_N), jnp.int32),
        scratch_types=[
            pltpu.VMEM((SC_ROWS, NB), jnp.int32),
            pltpu.VMEM((SC_ROWS, TOP_N), jnp.int32),
        ],
        compiler_params=pltpu.CompilerParams(needs_layout_passes=False),
    )


def kernel(q, k_buffer, v_buffer, Wk, Wv):
    kb = k_buffer.reshape(R, STRIDE, NUM_KV_HEADS, D)
    vb = v_buffer.reshape(R, STRIDE, NUM_KV_HEADS, D)

    ck, cv = pl.pallas_call(
        _compress_kernel,
        out_shape=[
            jax.ShapeDtypeStruct((NUM_KV_HEADS, R, D), jnp.float32),
            jax.ShapeDtypeStruct((NUM_KV_HEADS, R, D), jnp.float32),
        ],
    )(kb, vb, Wk, Wv)

    o, slc = pl.pallas_call(
        _attn_kernel,
        grid=(S // SB,),
        in_specs=[
            pl.BlockSpec((SB, NUM_Q_HEADS * D), lambda i: (i, 0)),
            pl.BlockSpec((NUM_KV_HEADS, CP, D), lambda i: (0, 0, 0)),
            pl.BlockSpec((NUM_KV_HEADS, CP, D), lambda i: (0, 0, 0)),
        ],
        out_specs=[
            pl.BlockSpec((SB, NUM_Q_HEADS * D), lambda i: (i, 0)),
            pl.BlockSpec((SB, NUM_KV_HEADS * NB), lambda i: (i, 0)),
        ],
        out_shape=[
            jax.ShapeDtypeStruct((S, NUM_Q_HEADS * D), jnp.float32),
            jax.ShapeDtypeStruct((S, NUM_KV_HEADS * NB), jnp.int32),
        ],
    )(q, ck, cv)

    topk = _make_topk_sc()(slc.reshape(S * NUM_KV_HEADS, NB))
    return o, topk.reshape(S, NUM_KV_HEADS, TOP_N)


# speedup vs baseline: 2.9717x; 1.1254x over previous
import jax
import jax.numpy as jnp
from jax import lax
from jax.experimental import pallas as pl
from jax.experimental.pallas import tpu as pltpu
from jax.experimental.pallas import tpu_sc as plsc

NUM_Q_HEADS = 32
NUM_KV_HEADS = 2
D = 128
KS = 32
STRIDE = 16
SELECT = 64
TOP_N = 16
N_INIT = 1
N_LOCAL = 2
GROUP = NUM_Q_HEADS // NUM_KV_HEADS
SM_SCALE = D ** (-0.5)
S = 4096
R = S // STRIDE
CP = 256
NB = S // SELECT
NEG = -1e30

SB = 512

SC_NC = 2
SC_NS = 16
SC_NW = SC_NC * SC_NS
SC_ROWS = (S * NUM_KV_HEADS) // SC_NW

BF = jnp.bfloat16


def _bdot(a, b):
    return lax.dot_general(a.astype(BF), b.astype(BF),
                           (((1,), (0,)), ((), ())),
                           preferred_element_type=jnp.float32)


def _compress_body(b_ref, w_ref, o_ref, h):
    xs = []
    g1 = jnp.zeros((R, KS), jnp.float32)
    g2 = jnp.zeros((R, KS), jnp.float32)
    for j in range(STRIDE):
        x = b_ref[:, j, h, :]
        xs.append(x)
        xb = x.astype(BF)
        w1j = w_ref[:, j * D:(j + 1) * D].astype(BF)
        w2j = w_ref[:, (j + STRIDE) * D:(j + STRIDE + 1) * D].astype(BF)
        g1 = g1 + lax.dot_general(xb, w1j, (((1,), (1,)), ((), ())),
                                  preferred_element_type=jnp.float32)
        g2 = g2 + lax.dot_general(xb, w2j, (((1,), (1,)), ((), ())),
                                  preferred_element_type=jnp.float32)
    gate = g1 + pltpu.roll(g2, R - 1, 0)
    mx = jnp.max(gate, axis=1, keepdims=True)
    e = jnp.exp(gate - mx)
    wts = e / jnp.sum(e, axis=1, keepdims=True)
    w2r = pltpu.roll(wts[:, STRIDE:], 1, 0)

    acc = jnp.zeros((R, D), jnp.float32)
    y = jnp.zeros((R, D), jnp.float32)
    for j in range(STRIDE):
        acc = acc + wts[:, j:j + 1] * xs[j]
        y = y + w2r[:, j:j + 1] * xs[j]
    o_ref[h] = acc + pltpu.roll(y, R - 1, 0)


def _compress_kernel(kb_ref, vb_ref, wk_ref, wv_ref, ck_ref, cv_ref):
    for h in range(NUM_KV_HEADS):
        _compress_body(kb_ref, wk_ref, ck_ref, h)
        _compress_body(vb_ref, wv_ref, cv_ref, h)


def _attn_kernel(q_ref, ck_ref, cv_ref, o_ref, slc_ref):
    i = pl.program_id(0)
    t = i * SB + lax.broadcasted_iota(jnp.int32, (SB, CP), 0)
    c = lax.broadcasted_iota(jnp.int32, (SB, CP), 1)
    valid = (t >= c * STRIDE + KS - 1) & (c < CP - 1)
    mask = valid | ((t < KS - 1) & (c == 0))

    ci = lax.broadcasted_iota(jnp.int32, (CP, NB), 0)
    bi = lax.broadcasted_iota(jnp.int32, (CP, NB), 1)
    agg = ((ci // (SELECT // STRIDE)) == bi).astype(jnp.float32)

    qb = (i * SB + lax.broadcasted_iota(jnp.int32, (SB, NB), 0)) // SELECT
    b2 = lax.broadcasted_iota(jnp.int32, (SB, NB), 1)
    forced = (b2 < N_INIT) | ((b2 <= qb) & (b2 >= qb - (N_LOCAL - 1)))
    causal = b2 <= qb
    b2f = b2.astype(jnp.float32)
    forced_vals = 1e9 - b2f * 1024.0
    masked_vals = NEG - b2f * 4e24

    for h in range(NUM_KV_HEADS):
        ckh = ck_ref[h].astype(BF)
        cvh = cv_ref[h].astype(BF)
        p = jnp.zeros((SB, CP), jnp.float32)
        for g in range(GROUP):
            hq = h * GROUP + g
            qg = q_ref[:, hq * D:(hq + 1) * D].astype(BF)
            s = lax.dot_general(qg, ckh, (((1,), (1,)), ((), ())),
                                preferred_element_type=jnp.float32)
            s = jnp.where(mask, s * SM_SCALE, NEG)
            m = jnp.max(s, axis=1, keepdims=True)
            e = jnp.exp(s - m)
            probs = e * (1.0 / jnp.sum(e, axis=1, keepdims=True))
            o_ref[:, hq * D:(hq + 1) * D] = _bdot(probs, cvh)
            p = p + probs
        slc = _bdot(p, agg)
        slc = jnp.where(forced, forced_vals, slc)
        slc = jnp.where(causal, slc, masked_vals)
        bits = pltpu.bitcast(slc, jnp.int32)
        okey = jnp.where(bits < 0, bits ^ jnp.int32(0x7FFFFFFF), bits)
        slc_ref[:, h * NB:(h + 1) * NB] = okey - b2


def _topk_sc_body(slc_hbm, out_hbm, slc_v, out_v):
    wid = lax.axis_index("s") * SC_NC + lax.axis_index("c")
    base = wid * SC_ROWS
    pltpu.sync_copy(slc_hbm.at[pl.ds(base, SC_ROWS)], slc_v)
    lane = lax.broadcasted_iota(jnp.int32, (16,), 0)

    def merge(a, b):
        ak, ai = a
        bk = lax.rev(b[0], (0,))
        bi = lax.rev(b[1], (0,))
        take_a = ak >= bk
        mk = jnp.where(take_a, ak, bk)
        mi = jnp.where(take_a, ai, bi)
        return plsc.sort_key_val(mk, mi, descending=True)

    def body(rr, _):
        runs = []
        for j in range(NB // 16):
            k = slc_v[rr, pl.ds(16 * j, 16)]
            runs.append(plsc.sort_key_val(k, lane + 16 * j, descending=True))
        mk, mi = merge(merge(runs[0], runs[1]), merge(runs[2], runs[3]))
        out_v[rr] = mi
        return 0

    lax.fori_loop(0, SC_ROWS, body, 0)
    pltpu.sync_copy(out_v, out_hbm.at[pl.ds(base, SC_ROWS)])


def _make_topk_sc():
    mesh = plsc.VectorSubcoreMesh(core_axis_name="c", subcore_axis_name="s")
    return pl.kernel(
        _topk_sc_body,
        mesh=mesh,
        out_type=jax.ShapeDtypeStruct((S * NUM_KV_HEADS, TOP_N), jnp.int32),
        scratch_types=[
            pltpu.VMEM((SC_ROWS, NB), jnp.int32),
            pltpu.VMEM((SC_ROWS, TOP_N), jnp.int32),
        ],
        compiler_params=pltpu.CompilerParams(needs_layout_passes=False),
    )


def kernel(q, k_buffer, v_buffer, Wk, Wv):
    kb = k_buffer.reshape(R, STRIDE, NUM_KV_HEADS, D)
    vb = v_buffer.reshape(R, STRIDE, NUM_KV_HEADS, D)

    ck, cv = pl.pallas_call(
        _compress_kernel,
        out_shape=[
            jax.ShapeDtypeStruct((NUM_KV_HEADS, R, D), jnp.float32),
            jax.ShapeDtypeStruct((NUM_KV_HEADS, R, D), jnp.float32),
        ],
    )(kb, vb, Wk, Wv)

    o, slc = pl.pallas_call(
        _attn_kernel,
        grid=(S // SB,),
        in_specs=[
            pl.BlockSpec((SB, NUM_Q_HEADS * D), lambda i: (i, 0)),
            pl.BlockSpec((NUM_KV_HEADS, CP, D), lambda i: (0, 0, 0)),
            pl.BlockSpec((NUM_KV_HEADS, CP, D), lambda i: (0, 0, 0)),
        ],
        out_specs=[
            pl.BlockSpec((SB, NUM_Q_HEADS * D), lambda i: (i, 0)),
            pl.BlockSpec((SB, NUM_KV_HEADS * NB), lambda i: (i, 0)),
        ],
        out_shape=[
            jax.ShapeDtypeStruct((S, NUM_Q_HEADS * D), jnp.float32),
            jax.ShapeDtypeStruct((S, NUM_KV_HEADS * NB), jnp.int32),
        ],
    )(q, ck, cv)

    topk = _make_topk_sc()(slc.reshape(S * NUM_KV_HEADS, NB))
    return o, topk.reshape(S, NUM_KV_HEADS, TOP_N)
